# Initial kernel scaffold; baseline (speedup 1.0000x reference)
import jax
import jax.numpy as jnp
from jax.experimental import pallas as pl


def kernel(x, edge_index, fc0_W, fc0_b, conv_weight, conv_watt, fc1_W, fc1_b):
    raise NotImplementedError("write your pallas kernel here")


# speedup vs baseline: 1.3243x; 1.3243x over previous
import functools
import math

import jax
import jax.numpy as jnp
from jax.experimental import pallas as pl
from jax.experimental.pallas import tpu as pltpu

N = 10000
E = 320000
NFEAT = 128
NH = 64
NCLASS = 40
NLAYERS = 2
MOMENT = 3
LAMDA = 0.5
ALPHA = 0.1

BN = 1000


def _pre_body(x_ref, w_ref, b_ref, h_ref):
    h = jnp.dot(x_ref[...], w_ref[...], preferred_element_type=jnp.float32)
    h_ref[...] = jnp.maximum(h + b_ref[...], 0.0)


def _pre_kernel(x, fc0_W, fc0_b):
    return pl.pallas_call(
        _pre_body,
        grid=(N // BN,),
        in_specs=[
            pl.BlockSpec((BN, NFEAT), lambda i: (i, 0)),
            pl.BlockSpec((NFEAT, NH), lambda i: (0, 0)),
            pl.BlockSpec((1, NH), lambda i: (0, 0)),
        ],
        out_specs=pl.BlockSpec((BN, NH), lambda i: (i, 0)),
        out_shape=jax.ShapeDtypeStruct((N, NH), jnp.float32),
    )(x, fc0_W, fc0_b.reshape(1, NH))


def _layer_body(theta, s1_ref, s2_ref, s3_ref, h_ref, h0_ref, wa_ref, w_ref,
                out_ref):
    mu = s1_ref[...]
    sigma = jnp.sqrt(jnp.clip(s2_ref[...] - mu * mu, 1e-6, None))
    raw3 = s3_ref[...]
    m3 = jnp.sign(raw3) * jnp.power(jnp.abs(raw3) + 1e-6, 1.0 / 3.0)
    h = h_ref[...]
    wa = wa_ref[...]
    waQ = wa[:, :NH]
    waK = wa[:, NH:]
    q = jnp.dot(h, waQ.T, preferred_element_type=jnp.float32)
    k0 = jnp.dot(mu, waK[0:1, :].T, preferred_element_type=jnp.float32)
    k1 = jnp.dot(sigma, waK[1:2, :].T, preferred_element_type=jnp.float32)
    k2 = jnp.dot(m3, waK[2:3, :].T, preferred_element_type=jnp.float32)
    k = jnp.concatenate([k0, k1, k2], axis=1)
    a = jnp.tanh(q + k)
    a = a - jnp.max(a, axis=1, keepdims=True)
    ea = jnp.exp(a)
    attn = ea / jnp.sum(ea, axis=1, keepdims=True)
    agg = (attn[:, 0:1] * mu + attn[:, 1:2] * sigma + attn[:, 2:3] * m3)
    support = (1.0 - ALPHA) * agg + ALPHA * h0_ref[...]
    z = jnp.dot(support, w_ref[...], preferred_element_type=jnp.float32)
    out_ref[...] = jnp.maximum(theta * z + (1.0 - theta) * support, 0.0)


def _layer_kernel(l, s1, s2, s3, h, h0, wa, w):
    theta = math.log(LAMDA / (l + 1) + 1.0)
    blk = pl.BlockSpec((BN, NH), lambda i: (i, 0))
    return pl.pallas_call(
        functools.partial(_layer_body, theta),
        grid=(N // BN,),
        in_specs=[
            blk, blk, blk, blk, blk,
            pl.BlockSpec((MOMENT, 2 * NH), lambda i: (0, 0)),
            pl.BlockSpec((NH, NH), lambda i: (0, 0)),
        ],
        out_specs=blk,
        out_shape=jax.ShapeDtypeStruct((N, NH), jnp.float32),
    )(s1, s2, s3, h, h0, wa, w)


def _final_body(h_ref, w_ref, b_ref, out_ref):
    z = jnp.dot(h_ref[...], w_ref[...], preferred_element_type=jnp.float32)
    z = z + b_ref[...]
    z = z - jnp.max(z, axis=1, keepdims=True)
    out_ref[...] = z - jnp.log(jnp.sum(jnp.exp(z), axis=1, keepdims=True))


def _final_kernel(h, fc1_W, fc1_b):
    return pl.pallas_call(
        _final_body,
        grid=(N // BN,),
        in_specs=[
            pl.BlockSpec((BN, NH), lambda i: (i, 0)),
            pl.BlockSpec((NH, NCLASS), lambda i: (0, 0)),
            pl.BlockSpec((1, NCLASS), lambda i: (0, 0)),
        ],
        out_specs=pl.BlockSpec((BN, NCLASS), lambda i: (i, 0)),
        out_shape=jax.ShapeDtypeStruct((N, NCLASS), jnp.float32),
    )(h, fc1_W, fc1_b.reshape(1, NCLASS))


def kernel(x, edge_index, fc0_W, fc0_b, conv_weight, conv_watt, fc1_W, fc1_b):
    src = edge_index[0]
    dst = edge_index[1]
    deg = jnp.zeros((N,), jnp.float32).at[src].add(1.0).at[dst].add(1.0)
    rs = jax.lax.rsqrt(jnp.clip(deg, 1.0, None))
    w = rs[src] * rs[dst]

    def spmm3(h):
        g = h[src]
        wg = w[:, None]
        z = jnp.zeros((N, NH), jnp.float32)
        s1 = z.at[dst].add(wg * g)
        s2 = z.at[dst].add(wg * (g * g))
        s3 = z.at[dst].add(wg * (g * g * g))
        return s1, s2, s3

    h = _pre_kernel(x, fc0_W, fc0_b)
    h0 = h
    for l in range(NLAYERS):
        s1, s2, s3 = spmm3(h)
        h = _layer_kernel(l, s1, s2, s3, h, h0, conv_watt[l], conv_weight[l])
    return _final_kernel(h, fc1_W, fc1_b)


# speedup vs baseline: 11.3490x; 8.5695x over previous
import functools
import math

import jax
import jax.numpy as jnp
from jax import lax
from jax.experimental import pallas as pl
from jax.experimental.pallas import tpu as pltpu
from jax.experimental.pallas import tpu_sc as plsc

N = 10000
E = 320000
NFEAT = 128
NH = 64
NCLASS = 40
NLAYERS = 2
MOMENT = 3
LAMDA = 0.5
ALPHA = 0.1

NC = 2
NS = 16
L = 16
NW = NC * NS

BN = 1000
EPW = E // NW
EB = 16000
PACK = 16384

_sc_mesh = plsc.VectorSubcoreMesh(core_axis_name="c", subcore_axis_name="s")


def _wid():
    return lax.axis_index("s") * NC + lax.axis_index("c")


def _pack_body(e_ref, out_ref):
    out_ref[...] = e_ref[1:2, :] * PACK + e_ref[0:1, :]


def _pack_kernel(edge_index):
    eb = 32000
    return pl.pallas_call(
        _pack_body,
        grid=(E // eb,),
        in_specs=[pl.BlockSpec((2, eb), lambda i: (0, i))],
        out_specs=pl.BlockSpec((1, eb), lambda i: (0, i)),
        out_shape=jax.ShapeDtypeStruct((1, E), jnp.int32),
    )(edge_index)


def _deg_body(packed_hbm, degp_hbm, ebuf, acc):
    wid = _wid()
    z16 = jnp.zeros((L,), jnp.float32)

    def zbody(i, c):
        acc[pl.ds(i * L, L)] = z16
        return c

    lax.fori_loop(0, N // L, zbody, 0)
    pltpu.sync_copy(packed_hbm.at[pl.ds(wid * EPW, EPW)], ebuf)
    ones = jnp.ones((L,), jnp.float32)

    def body(i, c):
        p = ebuf[pl.ds(i * L, L)]
        s = jnp.bitwise_and(p, PACK - 1)
        d = jnp.right_shift(p, 14)
        plsc.addupdate_scatter(acc, [s], ones)
        plsc.addupdate_scatter(acc, [d], ones)
        return c

    lax.fori_loop(0, EPW // L, body, 0)
    pltpu.sync_copy(acc, degp_hbm.at[pl.ds(wid * N, N)])


_deg_kernel = functools.partial(
    pl.kernel,
    out_type=jax.ShapeDtypeStruct((NW * N,), jnp.float32),
    mesh=_sc_mesh,
    compiler_params=pltpu.CompilerParams(needs_layout_passes=False),
    scratch_types=[
        pltpu.VMEM((EPW,), jnp.int32),
        pltpu.VMEM((N,), jnp.float32),
    ],
)(_deg_body)


def _mom_body(hT_hbm, packed_hbm, rs_hbm, out_hbm, ebuf, hc0, hc1, rsv, acc):
    wid = _wid()
    c0 = 2 * wid
    pltpu.sync_copy(hT_hbm.at[pl.ds(c0 * N, N)], hc0)
    pltpu.sync_copy(hT_hbm.at[pl.ds((c0 + 1) * N, N)], hc1)
    pltpu.sync_copy(rs_hbm, rsv)
    z16 = jnp.zeros((L,), jnp.float32)

    def zbody(i, c):
        acc[pl.ds(i * L, L)] = z16
        return c

    lax.fori_loop(0, 6 * N // L, zbody, 0)

    def body(i, c):
        p = ebuf[pl.ds(i * L, L)]
        s = jnp.bitwise_and(p, PACK - 1)
        d = jnp.right_shift(p, 14)
        w = plsc.load_gather(rsv, [s]) * plsc.load_gather(rsv, [d])
        g0 = plsc.load_gather(hc0, [s])
        g1 = plsc.load_gather(hc1, [s])
        a0 = w * g0
        a1 = w * g1
        plsc.addupdate_scatter(acc, [d], a0)
        plsc.addupdate_scatter(acc, [d + N], a1)
        a0 = a0 * g0
        a1 = a1 * g1
        plsc.addupdate_scatter(acc, [d + 2 * N], a0)
        plsc.addupdate_scatter(acc, [d + 3 * N], a1)
        a0 = a0 * g0
        a1 = a1 * g1
        plsc.addupdate_scatter(acc, [d + 4 * N], a0)
        plsc.addupdate_scatter(acc, [d + 5 * N], a1)
        return c

    for blk in range(E // EB):
        pltpu.sync_copy(packed_hbm.at[pl.ds(blk * EB, EB)], ebuf)
        lax.fori_loop(0, EB // L, body, 0)

    for pw in range(3):
        pltpu.sync_copy(acc.at[pl.ds((2 * pw) * N, N)],
                        out_hbm.at[pl.ds((pw * NH + c0) * N, N)])
        pltpu.sync_copy(acc.at[pl.ds((2 * pw + 1) * N, N)],
                        out_hbm.at[pl.ds((pw * NH + c0 + 1) * N, N)])


_mom_kernel = functools.partial(
    pl.kernel,
    out_type=jax.ShapeDtypeStruct((3 * NH * N,), jnp.float32),
    mesh=_sc_mesh,
    compiler_params=pltpu.CompilerParams(needs_layout_passes=False),
    scratch_types=[
        pltpu.VMEM((EB,), jnp.int32),
        pltpu.VMEM((N,), jnp.float32),
        pltpu.VMEM((N,), jnp.float32),
        pltpu.VMEM((N,), jnp.float32),
        pltpu.VMEM((6 * N,), jnp.float32),
    ],
)(_mom_body)


def _pre_body(xT_ref, wt_ref, b_ref, degp_ref, hT_ref, rs_ref):
    deg = jnp.sum(degp_ref[...], axis=0, keepdims=True)
    rs_ref[...] = lax.rsqrt(jnp.clip(deg, 1.0, None))
    z = jnp.dot(wt_ref[...], xT_ref[...], preferred_element_type=jnp.float32)
    hT_ref[...] = jnp.maximum(z + b_ref[...], 0.0)


def _pre_kernel(xT, fc0_Wt, fc0_b, degp):
    return pl.pallas_call(
        _pre_body,
        out_shape=[
            jax.ShapeDtypeStruct((NH, N), jnp.float32),
            jax.ShapeDtypeStruct((1, N), jnp.float32),
        ],
    )(xT, fc0_Wt, fc0_b.reshape(NH, 1), degp)


def _layer_body(theta, momT_ref, hT_ref, h0T_ref, wa_ref, wt_ref, out_ref):
    momT = momT_ref[...]
    mu = momT[0:NH, :]
    s2 = momT[NH:2 * NH, :]
    s3 = momT[2 * NH:3 * NH, :]
    sigma = jnp.sqrt(jnp.clip(s2 - mu * mu, 1e-6, None))
    m3 = jnp.sign(s3) * jnp.power(jnp.abs(s3) + 1e-6, 1.0 / 3.0)
    hT = hT_ref[...]
    wa = wa_ref[...]
    waQ = wa[:, :NH]
    waK = wa[:, NH:]
    qT = jnp.dot(waQ, hT, preferred_element_type=jnp.float32)
    k0 = jnp.dot(waK[0:1, :], mu, preferred_element_type=jnp.float32)
    k1 = jnp.dot(waK[1:2, :], sigma, preferred_element_type=jnp.float32)
    k2 = jnp.dot(waK[2:3, :], m3, preferred_element_type=jnp.float32)
    a = jnp.tanh(qT + jnp.concatenate([k0, k1, k2], axis=0))
    a = a - jnp.max(a, axis=0, keepdims=True)
    ea = jnp.exp(a)
    attn = ea / jnp.sum(ea, axis=0, keepdims=True)
    aggT = attn[0:1, :] * mu + attn[1:2, :] * sigma + attn[2:3, :] * m3
    supportT = (1.0 - ALPHA) * aggT + ALPHA * h0T_ref[...]
    zT = jnp.dot(wt_ref[...], supportT, preferred_element_type=jnp.float32)
    out_ref[...] = jnp.maximum(theta * zT + (1.0 - theta) * supportT, 0.0)


def _layer_kernel(l, momT, hT, h0T, wa, wt):
    theta = math.log(LAMDA / (l + 1) + 1.0)
    return pl.pallas_call(
        functools.partial(_layer_body, theta),
        out_shape=jax.ShapeDtypeStruct((NH, N), jnp.float32),
    )(momT, hT, h0T, wa, wt)


def _final_body(hT_ref, wt_ref, b_ref, out_ref):
    zT = jnp.dot(wt_ref[...], hT_ref[...], preferred_element_type=jnp.float32)
    zT = zT + b_ref[...]
    zT = zT - jnp.max(zT, axis=0, keepdims=True)
    out_ref[...] = zT - jnp.log(jnp.sum(jnp.exp(zT), axis=0, keepdims=True))


def _final_kernel(hT, fc1_Wt, fc1_b):
    return pl.pallas_call(
        _final_body,
        out_shape=jax.ShapeDtypeStruct((NCLASS, N), jnp.float32),
    )(hT, fc1_Wt, fc1_b.reshape(NCLASS, 1))


def kernel(x, edge_index, fc0_W, fc0_b, conv_weight, conv_watt, fc1_W, fc1_b):
    packed = _pack_kernel(edge_index).reshape(E)
    degp = _deg_kernel(packed).reshape(NW, N)
    hT, rs = _pre_kernel(x.T, fc0_W.T, fc0_b, degp)
    h0T = hT
    rs_flat = rs.reshape(N)
    for l in range(NLAYERS):
        momT = _mom_kernel(hT.reshape(NH * N), packed, rs_flat)
        hT = _layer_kernel(l, momT.reshape(3 * NH, N), hT, h0T,
                           conv_watt[l], conv_weight[l].T)
    return _final_kernel(hT, fc1_W.T, fc1_b).T


# speedup vs baseline: 18.5316x; 1.6329x over previous
import functools
import math

import jax
import jax.numpy as jnp
from jax import lax
from jax.experimental import pallas as pl
from jax.experimental.pallas import tpu as pltpu
from jax.experimental.pallas import tpu_sc as plsc

N = 10000
E = 320000
NFEAT = 128
NH = 64
NCLASS = 40
NLAYERS = 2
MOMENT = 3
LAMDA = 0.5
ALPHA = 0.1

NC = 2
NS = 16
L = 16
NW = NC * NS

BN = 1000
EPW = E // NW
EB = 16000
PACK = 16384

_sc_mesh = plsc.VectorSubcoreMesh(core_axis_name="c", subcore_axis_name="s")


def _wid():
    return lax.axis_index("s") * NC + lax.axis_index("c")


def _pack_body(e_ref, out_ref):
    out_ref[...] = e_ref[1:2, :] * PACK + e_ref[0:1, :]


def _pack_kernel(edge_index):
    eb = 32000
    return pl.pallas_call(
        _pack_body,
        grid=(E // eb,),
        in_specs=[pl.BlockSpec((2, eb), lambda i: (0, i))],
        out_specs=pl.BlockSpec((1, eb), lambda i: (0, i)),
        out_shape=jax.ShapeDtypeStruct((1, E), jnp.int32),
    )(edge_index)


def _deg_body(packed_hbm, degp_hbm, ebuf, acc):
    wid = _wid()
    z16 = jnp.zeros((L,), jnp.float32)

    def zbody(i, c):
        acc[pl.ds(i * L, L)] = z16
        return c

    lax.fori_loop(0, N // L, zbody, 0)
    pltpu.sync_copy(packed_hbm.at[pl.ds(wid * EPW, EPW)], ebuf)
    ones = jnp.ones((L,), jnp.float32)

    def body(i, c):
        p = ebuf[pl.ds(i * L, L)]
        s = jnp.bitwise_and(p, PACK - 1)
        d = jnp.right_shift(p, 14)
        plsc.addupdate_scatter(acc, [s], ones)
        plsc.addupdate_scatter(acc, [d], ones)
        return c

    lax.fori_loop(0, EPW // L, body, 0)
    pltpu.sync_copy(acc, degp_hbm.at[pl.ds(wid * N, N)])


_deg_kernel = functools.partial(
    pl.kernel,
    out_type=jax.ShapeDtypeStruct((NW * N,), jnp.float32),
    mesh=_sc_mesh,
    compiler_params=pltpu.CompilerParams(needs_layout_passes=False),
    scratch_types=[
        pltpu.VMEM((EPW,), jnp.int32),
        pltpu.VMEM((N,), jnp.float32),
    ],
)(_deg_body)


def _mom_body(hT_hbm, t1T_hbm, packed_hbm, out_hbm, ebuf, hc0, hc1, tc0, tc1,
              acc):
    wid = _wid()
    c0 = 2 * wid
    pltpu.sync_copy(hT_hbm.at[pl.ds(c0 * N, N)], hc0)
    pltpu.sync_copy(hT_hbm.at[pl.ds((c0 + 1) * N, N)], hc1)
    pltpu.sync_copy(t1T_hbm.at[pl.ds(c0 * N, N)], tc0)
    pltpu.sync_copy(t1T_hbm.at[pl.ds((c0 + 1) * N, N)], tc1)
    z16 = jnp.zeros((L,), jnp.float32)

    @plsc.parallel_loop(0, 6 * N, step=L)
    def _(i):
        acc[pl.ds(i, L)] = z16

    def body(i):
        p = ebuf[pl.ds(i, L)]
        s = jnp.bitwise_and(p, PACK - 1)
        d = jnp.right_shift(p, 14)
        a0 = plsc.load_gather(tc0, [s])
        a1 = plsc.load_gather(tc1, [s])
        g0 = plsc.load_gather(hc0, [s])
        g1 = plsc.load_gather(hc1, [s])
        plsc.addupdate_scatter(acc, [d], a0)
        plsc.addupdate_scatter(acc, [d + N], a1)
        a0 = a0 * g0
        a1 = a1 * g1
        plsc.addupdate_scatter(acc, [d + 2 * N], a0)
        plsc.addupdate_scatter(acc, [d + 3 * N], a1)
        a0 = a0 * g0
        a1 = a1 * g1
        plsc.addupdate_scatter(acc, [d + 4 * N], a0)
        plsc.addupdate_scatter(acc, [d + 5 * N], a1)

    for blk in range(E // EB):
        pltpu.sync_copy(packed_hbm.at[pl.ds(blk * EB, EB)], ebuf)
        plsc.parallel_loop(0, EB, step=L, unroll=4)(body)

    for pw in range(3):
        pltpu.sync_copy(acc.at[pl.ds((2 * pw) * N, N)],
                        out_hbm.at[pl.ds((pw * NH + c0) * N, N)])
        pltpu.sync_copy(acc.at[pl.ds((2 * pw + 1) * N, N)],
                        out_hbm.at[pl.ds((pw * NH + c0 + 1) * N, N)])


_mom_kernel = functools.partial(
    pl.kernel,
    out_type=jax.ShapeDtypeStruct((3 * NH * N,), jnp.float32),
    mesh=_sc_mesh,
    compiler_params=pltpu.CompilerParams(needs_layout_passes=False),
    scratch_types=[
        pltpu.VMEM((EB,), jnp.int32),
        pltpu.VMEM((N,), jnp.float32),
        pltpu.VMEM((N,), jnp.float32),
        pltpu.VMEM((N,), jnp.float32),
        pltpu.VMEM((N,), jnp.float32),
        pltpu.VMEM((6 * N,), jnp.float32),
    ],
)(_mom_body)


def _pre_body(xT_ref, wt_ref, b_ref, degp_ref, hT_ref, t1T_ref, rs_ref):
    deg = jnp.sum(degp_ref[...], axis=0, keepdims=True)
    rs = lax.rsqrt(jnp.clip(deg, 1.0, None))
    rs_ref[...] = rs
    z = jnp.dot(wt_ref[...], xT_ref[...], preferred_element_type=jnp.float32)
    h = jnp.maximum(z + b_ref[...], 0.0)
    hT_ref[...] = h
    t1T_ref[...] = rs * h


def _pre_kernel(xT, fc0_Wt, fc0_b, degp):
    return pl.pallas_call(
        _pre_body,
        out_shape=[
            jax.ShapeDtypeStruct((NH, N), jnp.float32),
            jax.ShapeDtypeStruct((NH, N), jnp.float32),
            jax.ShapeDtypeStruct((1, N), jnp.float32),
        ],
    )(xT, fc0_Wt, fc0_b.reshape(NH, 1), degp)


def _layer_body(theta, momT_ref, rs_ref, hT_ref, h0T_ref, wa_ref, wt_ref,
                out_ref, t1T_ref):
    rs = rs_ref[...]
    momT = momT_ref[...] * rs
    mu = momT[0:NH, :]
    s2 = momT[NH:2 * NH, :]
    s3 = momT[2 * NH:3 * NH, :]
    sigma = jnp.sqrt(jnp.clip(s2 - mu * mu, 1e-6, None))
    m3 = jnp.sign(s3) * jnp.power(jnp.abs(s3) + 1e-6, 1.0 / 3.0)
    hT = hT_ref[...]
    wa = wa_ref[...]
    waQ = wa[:, :NH]
    waK = wa[:, NH:]
    qT = jnp.dot(waQ, hT, preferred_element_type=jnp.float32)
    k0 = jnp.dot(waK[0:1, :], mu, preferred_element_type=jnp.float32)
    k1 = jnp.dot(waK[1:2, :], sigma, preferred_element_type=jnp.float32)
    k2 = jnp.dot(waK[2:3, :], m3, preferred_element_type=jnp.float32)
    a = jnp.tanh(qT + jnp.concatenate([k0, k1, k2], axis=0))
    a = a - jnp.max(a, axis=0, keepdims=True)
    ea = jnp.exp(a)
    attn = ea / jnp.sum(ea, axis=0, keepdims=True)
    aggT = attn[0:1, :] * mu + attn[1:2, :] * sigma + attn[2:3, :] * m3
    supportT = (1.0 - ALPHA) * aggT + ALPHA * h0T_ref[...]
    zT = jnp.dot(wt_ref[...], supportT, preferred_element_type=jnp.float32)
    h = jnp.maximum(theta * zT + (1.0 - theta) * supportT, 0.0)
    out_ref[...] = h
    t1T_ref[...] = rs * h


def _layer_kernel(l, momT, rs, hT, h0T, wa, wt):
    theta = math.log(LAMDA / (l + 1) + 1.0)
    return pl.pallas_call(
        functools.partial(_layer_body, theta),
        out_shape=[
            jax.ShapeDtypeStruct((NH, N), jnp.float32),
            jax.ShapeDtypeStruct((NH, N), jnp.float32),
        ],
    )(momT, rs, hT, h0T, wa, wt)


def _final_body(hT_ref, wt_ref, b_ref, out_ref):
    zT = jnp.dot(wt_ref[...], hT_ref[...], preferred_element_type=jnp.float32)
    zT = zT + b_ref[...]
    zT = zT - jnp.max(zT, axis=0, keepdims=True)
    out_ref[...] = zT - jnp.log(jnp.sum(jnp.exp(zT), axis=0, keepdims=True))


def _final_kernel(hT, fc1_Wt, fc1_b):
    return pl.pallas_call(
        _final_body,
        out_shape=jax.ShapeDtypeStruct((NCLASS, N), jnp.float32),
    )(hT, fc1_Wt, fc1_b.reshape(NCLASS, 1))


def kernel(x, edge_index, fc0_W, fc0_b, conv_weight, conv_watt, fc1_W, fc1_b):
    packed = _pack_kernel(edge_index).reshape(E)
    degp = _deg_kernel(packed).reshape(NW, N)
    hT, t1T, rs = _pre_kernel(x.T, fc0_W.T, fc0_b, degp)
    h0T = hT
    for l in range(NLAYERS):
        momT = _mom_kernel(hT.reshape(NH * N), t1T.reshape(NH * N), packed)
        hT, t1T = _layer_kernel(l, momT.reshape(3 * NH, N), rs, hT, h0T,
                                conv_watt[l], conv_weight[l].T)
    return _final_kernel(hT, fc1_W.T, fc1_b).T


# speedup vs baseline: 18.9156x; 1.0207x over previous
import functools
import math

import jax
import jax.numpy as jnp
from jax import lax
from jax.experimental import pallas as pl
from jax.experimental.pallas import tpu as pltpu
from jax.experimental.pallas import tpu_sc as plsc

N = 10000
E = 320000
NFEAT = 128
NH = 64
NCLASS = 40
NLAYERS = 2
MOMENT = 3
LAMDA = 0.5
ALPHA = 0.1

NC = 2
NS = 16
L = 16
NW = NC * NS

BN = 1000
EPW = E // NW
EB = 20000
PACK = 16384

_sc_mesh = plsc.VectorSubcoreMesh(core_axis_name="c", subcore_axis_name="s")


def _wid():
    return lax.axis_index("s") * NC + lax.axis_index("c")


def _pack_body(e_ref, out_ref):
    out_ref[...] = e_ref[1:2, :] * PACK + e_ref[0:1, :]


def _pack_kernel(edge_index):
    eb = 32000
    return pl.pallas_call(
        _pack_body,
        grid=(E // eb,),
        in_specs=[pl.BlockSpec((2, eb), lambda i: (0, i))],
        out_specs=pl.BlockSpec((1, eb), lambda i: (0, i)),
        out_shape=jax.ShapeDtypeStruct((1, E), jnp.int32),
    )(edge_index)


def _deg_body(packed_hbm, degp_hbm, ebuf, acc):
    wid = _wid()
    z16 = jnp.zeros((L,), jnp.float32)

    def zbody(i, c):
        acc[pl.ds(i * L, L)] = z16
        return c

    lax.fori_loop(0, N // L, zbody, 0)
    pltpu.sync_copy(packed_hbm.at[pl.ds(wid * EPW, EPW)], ebuf)
    ones = jnp.ones((L,), jnp.float32)

    def body(i, c):
        p = ebuf[pl.ds(i * L, L)]
        s = jnp.bitwise_and(p, PACK - 1)
        d = jnp.right_shift(p, 14)
        plsc.addupdate_scatter(acc, [s], ones)
        plsc.addupdate_scatter(acc, [d], ones)
        return c

    lax.fori_loop(0, EPW // L, body, 0)
    pltpu.sync_copy(acc, degp_hbm.at[pl.ds(wid * N, N)])


_deg_kernel = functools.partial(
    pl.kernel,
    out_type=jax.ShapeDtypeStruct((NW * N,), jnp.float32),
    mesh=_sc_mesh,
    compiler_params=pltpu.CompilerParams(needs_layout_passes=False),
    scratch_types=[
        pltpu.VMEM((EPW,), jnp.int32),
        pltpu.VMEM((N,), jnp.float32),
    ],
)(_deg_body)


def _mom_body(hT_hbm, t1T_hbm, packed_hbm, out_hbm, ebuf, hc0, hc1, tc0, tc1,
              acc):
    wid = _wid()
    c0 = 2 * wid
    pltpu.sync_copy(hT_hbm.at[pl.ds(c0 * N, N)], hc0)
    pltpu.sync_copy(hT_hbm.at[pl.ds((c0 + 1) * N, N)], hc1)
    pltpu.sync_copy(t1T_hbm.at[pl.ds(c0 * N, N)], tc0)
    pltpu.sync_copy(t1T_hbm.at[pl.ds((c0 + 1) * N, N)], tc1)
    z16 = jnp.zeros((L,), jnp.float32)

    @plsc.parallel_loop(0, 6 * N, step=L)
    def _(i):
        acc[pl.ds(i, L)] = z16

    def body(i):
        p = ebuf[pl.ds(i, L)]
        s = jnp.bitwise_and(p, PACK - 1)
        d = jnp.right_shift(p, 14)
        a0 = plsc.load_gather(tc0, [s])
        a1 = plsc.load_gather(tc1, [s])
        g0 = plsc.load_gather(hc0, [s])
        g1 = plsc.load_gather(hc1, [s])
        plsc.addupdate_scatter(acc, [d], a0)
        plsc.addupdate_scatter(acc, [d + N], a1)
        a0 = a0 * g0
        a1 = a1 * g1
        plsc.addupdate_scatter(acc, [d + 2 * N], a0)
        plsc.addupdate_scatter(acc, [d + 3 * N], a1)
        a0 = a0 * g0
        a1 = a1 * g1
        plsc.addupdate_scatter(acc, [d + 4 * N], a0)
        plsc.addupdate_scatter(acc, [d + 5 * N], a1)

    for blk in range(E // EB):
        pltpu.sync_copy(packed_hbm.at[pl.ds(blk * EB, EB)], ebuf)
        plsc.parallel_loop(0, EB, step=L, unroll=8)(body)

    for pw in range(3):
        pltpu.sync_copy(acc.at[pl.ds((2 * pw) * N, N)],
                        out_hbm.at[pl.ds((pw * NH + c0) * N, N)])
        pltpu.sync_copy(acc.at[pl.ds((2 * pw + 1) * N, N)],
                        out_hbm.at[pl.ds((pw * NH + c0 + 1) * N, N)])


_mom_kernel = functools.partial(
    pl.kernel,
    out_type=jax.ShapeDtypeStruct((3 * NH * N,), jnp.float32),
    mesh=_sc_mesh,
    compiler_params=pltpu.CompilerParams(needs_layout_passes=False),
    scratch_types=[
        pltpu.VMEM((EB,), jnp.int32),
        pltpu.VMEM((N,), jnp.float32),
        pltpu.VMEM((N,), jnp.float32),
        pltpu.VMEM((N,), jnp.float32),
        pltpu.VMEM((N,), jnp.float32),
        pltpu.VMEM((6 * N,), jnp.float32),
    ],
)(_mom_body)


def _pre_body(xT_ref, wt_ref, b_ref, degp_ref, hT_ref, t1T_ref, rs_ref):
    deg = jnp.sum(degp_ref[...], axis=0, keepdims=True)
    rs = lax.rsqrt(jnp.clip(deg, 1.0, None))
    rs_ref[...] = rs
    z = jnp.dot(wt_ref[...], xT_ref[...], preferred_element_type=jnp.float32)
    h = jnp.maximum(z + b_ref[...], 0.0)
    hT_ref[...] = h
    t1T_ref[...] = rs * h


def _pre_kernel(xT, fc0_Wt, fc0_b, degp):
    return pl.pallas_call(
        _pre_body,
        out_shape=[
            jax.ShapeDtypeStruct((NH, N), jnp.float32),
            jax.ShapeDtypeStruct((NH, N), jnp.float32),
            jax.ShapeDtypeStruct((1, N), jnp.float32),
        ],
    )(xT, fc0_Wt, fc0_b.reshape(NH, 1), degp)


def _layer_body(theta, momT_ref, rs_ref, hT_ref, h0T_ref, wa_ref, wt_ref,
                out_ref, t1T_ref):
    rs = rs_ref[...]
    momT = momT_ref[...] * rs
    mu = momT[0:NH, :]
    s2 = momT[NH:2 * NH, :]
    s3 = momT[2 * NH:3 * NH, :]
    sigma = jnp.sqrt(jnp.clip(s2 - mu * mu, 1e-6, None))
    m3 = jnp.sign(s3) * jnp.power(jnp.abs(s3) + 1e-6, 1.0 / 3.0)
    hT = hT_ref[...]
    wa = wa_ref[...]
    waQ = wa[:, :NH]
    waK = wa[:, NH:]
    qT = jnp.dot(waQ, hT, preferred_element_type=jnp.float32)
    k0 = jnp.dot(waK[0:1, :], mu, preferred_element_type=jnp.float32)
    k1 = jnp.dot(waK[1:2, :], sigma, preferred_element_type=jnp.float32)
    k2 = jnp.dot(waK[2:3, :], m3, preferred_element_type=jnp.float32)
    a = jnp.tanh(qT + jnp.concatenate([k0, k1, k2], axis=0))
    a = a - jnp.max(a, axis=0, keepdims=True)
    ea = jnp.exp(a)
    attn = ea / jnp.sum(ea, axis=0, keepdims=True)
    aggT = attn[0:1, :] * mu + attn[1:2, :] * sigma + attn[2:3, :] * m3
    supportT = (1.0 - ALPHA) * aggT + ALPHA * h0T_ref[...]
    zT = jnp.dot(wt_ref[...], supportT, preferred_element_type=jnp.float32)
    h = jnp.maximum(theta * zT + (1.0 - theta) * supportT, 0.0)
    out_ref[...] = h
    t1T_ref[...] = rs * h


def _layer_kernel(l, momT, rs, hT, h0T, wa, wt):
    theta = math.log(LAMDA / (l + 1) + 1.0)
    return pl.pallas_call(
        functools.partial(_layer_body, theta),
        out_shape=[
            jax.ShapeDtypeStruct((NH, N), jnp.float32),
            jax.ShapeDtypeStruct((NH, N), jnp.float32),
        ],
    )(momT, rs, hT, h0T, wa, wt)


def _final_body(hT_ref, wt_ref, b_ref, out_ref):
    zT = jnp.dot(wt_ref[...], hT_ref[...], preferred_element_type=jnp.float32)
    zT = zT + b_ref[...]
    zT = zT - jnp.max(zT, axis=0, keepdims=True)
    out_ref[...] = zT - jnp.log(jnp.sum(jnp.exp(zT), axis=0, keepdims=True))


def _final_kernel(hT, fc1_Wt, fc1_b):
    return pl.pallas_call(
        _final_body,
        out_shape=jax.ShapeDtypeStruct((NCLASS, N), jnp.float32),
    )(hT, fc1_Wt, fc1_b.reshape(NCLASS, 1))


def kernel(x, edge_index, fc0_W, fc0_b, conv_weight, conv_watt, fc1_W, fc1_b):
    packed = _pack_kernel(edge_index).reshape(E)
    degp = _deg_kernel(packed).reshape(NW, N)
    hT, t1T, rs = _pre_kernel(x.T, fc0_W.T, fc0_b, degp)
    h0T = hT
    for l in range(NLAYERS):
        momT = _mom_kernel(hT.reshape(NH * N), t1T.reshape(NH * N), packed)
        hT, t1T = _layer_kernel(l, momT.reshape(3 * NH, N), rs, hT, h0T,
                                conv_watt[l], conv_weight[l].T)
    return _final_kernel(hT, fc1_W.T, fc1_b).T


# speedup vs baseline: 20.6467x; 1.0915x over previous
import functools
import math

import jax
import jax.numpy as jnp
from jax import lax
from jax.experimental import pallas as pl
from jax.experimental.pallas import tpu as pltpu
from jax.experimental.pallas import tpu_sc as plsc

N = 10000
E = 320000
NFEAT = 128
NH = 64
NCLASS = 40
NLAYERS = 2
MOMENT = 3
LAMDA = 0.5
ALPHA = 0.1

NC = 2
NS = 16
L = 16
NW = NC * NS

BN = 1000
EPW = E // NW
EB = 10000
PACK = 16384

_sc_mesh = plsc.VectorSubcoreMesh(core_axis_name="c", subcore_axis_name="s")


def _wid():
    return lax.axis_index("s") * NC + lax.axis_index("c")


def _deg_body(edges_hbm, degp_hbm, packed_hbm, sbuf, dbuf, acc):
    wid = _wid()
    z16 = jnp.zeros((L,), jnp.float32)

    def zbody(i, c):
        acc[pl.ds(i * L, L)] = z16
        return c

    lax.fori_loop(0, N // L, zbody, 0)
    pltpu.sync_copy(edges_hbm.at[pl.ds(wid * EPW, EPW)], sbuf)
    pltpu.sync_copy(edges_hbm.at[pl.ds(E + wid * EPW, EPW)], dbuf)
    ones = jnp.ones((L,), jnp.float32)

    @plsc.parallel_loop(0, EPW, step=L)
    def _(i):
        s = sbuf[pl.ds(i, L)]
        d = dbuf[pl.ds(i, L)]
        plsc.addupdate_scatter(acc, [s], ones)
        plsc.addupdate_scatter(acc, [d], ones)
        sbuf[pl.ds(i, L)] = d * PACK + s

    pltpu.sync_copy(sbuf, packed_hbm.at[pl.ds(wid * EPW, EPW)])
    pltpu.sync_copy(acc, degp_hbm.at[pl.ds(wid * N, N)])


_deg_kernel = functools.partial(
    pl.kernel,
    out_type=[
        jax.ShapeDtypeStruct((NW * N,), jnp.float32),
        jax.ShapeDtypeStruct((E,), jnp.int32),
    ],
    mesh=_sc_mesh,
    compiler_params=pltpu.CompilerParams(needs_layout_passes=False),
    scratch_types=[
        pltpu.VMEM((EPW,), jnp.int32),
        pltpu.VMEM((EPW,), jnp.int32),
        pltpu.VMEM((N,), jnp.float32),
    ],
)(_deg_body)


def _mom_body(hT_hbm, t1T_hbm, packed_hbm, out_hbm, eb0, eb1, hc0, hc1, tc0,
              tc1, acc, sem0, sem1):
    wid = _wid()
    c0 = 2 * wid
    pltpu.sync_copy(hT_hbm.at[pl.ds(c0 * N, N)], hc0)
    pltpu.sync_copy(hT_hbm.at[pl.ds((c0 + 1) * N, N)], hc1)
    pltpu.sync_copy(t1T_hbm.at[pl.ds(c0 * N, N)], tc0)
    pltpu.sync_copy(t1T_hbm.at[pl.ds((c0 + 1) * N, N)], tc1)
    z16 = jnp.zeros((L,), jnp.float32)

    @plsc.parallel_loop(0, 6 * N, step=L)
    def _(i):
        acc[pl.ds(i, L)] = z16

    def make_body(ebuf):
        def body(i):
            p = ebuf[pl.ds(i, L)]
            s = jnp.bitwise_and(p, PACK - 1)
            d = jnp.right_shift(p, 14)
            a0 = plsc.load_gather(tc0, [s])
            a1 = plsc.load_gather(tc1, [s])
            g0 = plsc.load_gather(hc0, [s])
            g1 = plsc.load_gather(hc1, [s])
            plsc.addupdate_scatter(acc, [d], a0)
            plsc.addupdate_scatter(acc, [d + N], a1)
            a0 = a0 * g0
            a1 = a1 * g1
            plsc.addupdate_scatter(acc, [d + 2 * N], a0)
            plsc.addupdate_scatter(acc, [d + 3 * N], a1)
            a0 = a0 * g0
            a1 = a1 * g1
            plsc.addupdate_scatter(acc, [d + 4 * N], a0)
            plsc.addupdate_scatter(acc, [d + 5 * N], a1)

        return body

    bufs = (eb0, eb1)
    sems = (sem0, sem1)
    nblk = E // EB
    copies = [None, None]
    copies[0] = pltpu.async_copy(packed_hbm.at[pl.ds(0, EB)], bufs[0], sems[0])
    for blk in range(nblk):
        b = blk % 2
        copies[b].wait()
        if blk + 1 < nblk:
            copies[1 - b] = pltpu.async_copy(
                packed_hbm.at[pl.ds((blk + 1) * EB, EB)], bufs[1 - b],
                sems[1 - b])
        plsc.parallel_loop(0, EB, step=L, unroll=8)(make_body(bufs[b]))

    for pw in range(3):
        pltpu.sync_copy(acc.at[pl.ds((2 * pw) * N, N)],
                        out_hbm.at[pl.ds((pw * NH + c0) * N, N)])
        pltpu.sync_copy(acc.at[pl.ds((2 * pw + 1) * N, N)],
                        out_hbm.at[pl.ds((pw * NH + c0 + 1) * N, N)])


_mom_kernel = functools.partial(
    pl.kernel,
    out_type=jax.ShapeDtypeStruct((3 * NH * N,), jnp.float32),
    mesh=_sc_mesh,
    compiler_params=pltpu.CompilerParams(needs_layout_passes=False),
    scratch_types=[
        pltpu.VMEM((EB,), jnp.int32),
        pltpu.VMEM((EB,), jnp.int32),
        pltpu.VMEM((N,), jnp.float32),
        pltpu.VMEM((N,), jnp.float32),
        pltpu.VMEM((N,), jnp.float32),
        pltpu.VMEM((N,), jnp.float32),
        pltpu.VMEM((6 * N,), jnp.float32),
        pltpu.SemaphoreType.DMA,
        pltpu.SemaphoreType.DMA,
    ],
)(_mom_body)


def _pre_body(xT_ref, wt_ref, b_ref, degp_ref, hT_ref, t1T_ref, rs_ref):
    deg = jnp.sum(degp_ref[...], axis=0, keepdims=True)
    rs = lax.rsqrt(jnp.clip(deg, 1.0, None))
    rs_ref[...] = rs
    z = jnp.dot(wt_ref[...], xT_ref[...], preferred_element_type=jnp.float32)
    h = jnp.maximum(z + b_ref[...], 0.0)
    hT_ref[...] = h
    t1T_ref[...] = rs * h


def _pre_kernel(xT, fc0_Wt, fc0_b, degp):
    return pl.pallas_call(
        _pre_body,
        out_shape=[
            jax.ShapeDtypeStruct((NH, N), jnp.float32),
            jax.ShapeDtypeStruct((NH, N), jnp.float32),
            jax.ShapeDtypeStruct((1, N), jnp.float32),
        ],
    )(xT, fc0_Wt, fc0_b.reshape(NH, 1), degp)


def _layer_body(theta, momT_ref, rs_ref, hT_ref, h0T_ref, wa_ref, wt_ref,
                out_ref, t1T_ref):
    rs = rs_ref[...]
    momT = momT_ref[...] * rs
    mu = momT[0:NH, :]
    s2 = momT[NH:2 * NH, :]
    s3 = momT[2 * NH:3 * NH, :]
    sigma = jnp.sqrt(jnp.clip(s2 - mu * mu, 1e-6, None))
    m3 = jnp.sign(s3) * jnp.power(jnp.abs(s3) + 1e-6, 1.0 / 3.0)
    hT = hT_ref[...]
    wa = wa_ref[...]
    waQ = wa[:, :NH]
    waK = wa[:, NH:]
    qT = jnp.dot(waQ, hT, preferred_element_type=jnp.float32)
    k0 = jnp.dot(waK[0:1, :], mu, preferred_element_type=jnp.float32)
    k1 = jnp.dot(waK[1:2, :], sigma, preferred_element_type=jnp.float32)
    k2 = jnp.dot(waK[2:3, :], m3, preferred_element_type=jnp.float32)
    a = jnp.tanh(qT + jnp.concatenate([k0, k1, k2], axis=0))
    a = a - jnp.max(a, axis=0, keepdims=True)
    ea = jnp.exp(a)
    attn = ea / jnp.sum(ea, axis=0, keepdims=True)
    aggT = attn[0:1, :] * mu + attn[1:2, :] * sigma + attn[2:3, :] * m3
    supportT = (1.0 - ALPHA) * aggT + ALPHA * h0T_ref[...]
    zT = jnp.dot(wt_ref[...], supportT, preferred_element_type=jnp.float32)
    h = jnp.maximum(theta * zT + (1.0 - theta) * supportT, 0.0)
    out_ref[...] = h
    t1T_ref[...] = rs * h


def _layer_kernel(l, momT, rs, hT, h0T, wa, wt):
    theta = math.log(LAMDA / (l + 1) + 1.0)
    return pl.pallas_call(
        functools.partial(_layer_body, theta),
        out_shape=[
            jax.ShapeDtypeStruct((NH, N), jnp.float32),
            jax.ShapeDtypeStruct((NH, N), jnp.float32),
        ],
    )(momT, rs, hT, h0T, wa, wt)


def _final_body(hT_ref, wt_ref, b_ref, out_ref):
    zT = jnp.dot(wt_ref[...], hT_ref[...], preferred_element_type=jnp.float32)
    zT = zT + b_ref[...]
    zT = zT - jnp.max(zT, axis=0, keepdims=True)
    out_ref[...] = zT - jnp.log(jnp.sum(jnp.exp(zT), axis=0, keepdims=True))


def _final_kernel(hT, fc1_Wt, fc1_b):
    return pl.pallas_call(
        _final_body,
        out_shape=jax.ShapeDtypeStruct((NCLASS, N), jnp.float32),
    )(hT, fc1_Wt, fc1_b.reshape(NCLASS, 1))


def kernel(x, edge_index, fc0_W, fc0_b, conv_weight, conv_watt, fc1_W, fc1_b):
    degp_flat, packed = _deg_kernel(edge_index.reshape(2 * E))
    degp = degp_flat.reshape(NW, N)
    hT, t1T, rs = _pre_kernel(x.T, fc0_W.T, fc0_b, degp)
    h0T = hT
    for l in range(NLAYERS):
        momT = _mom_kernel(hT.reshape(NH * N), t1T.reshape(NH * N), packed)
        hT, t1T = _layer_kernel(l, momT.reshape(3 * NH, N), rs, hT, h0T,
                                conv_watt[l], conv_weight[l].T)
    return _final_kernel(hT, fc1_W.T, fc1_b).T


# speedup vs baseline: 20.7957x; 1.0072x over previous
import functools
import math

import jax
import jax.numpy as jnp
from jax import lax
from jax.experimental import pallas as pl
from jax.experimental.pallas import tpu as pltpu
from jax.experimental.pallas import tpu_sc as plsc

N = 10000
E = 320000
NFEAT = 128
NH = 64
NCLASS = 40
NLAYERS = 2
MOMENT = 3
LAMDA = 0.5
ALPHA = 0.1

NC = 2
NS = 16
L = 16
NW = NC * NS

BN = 1000
EPW = E // NW
EB = 10000
PACK = 16384

_sc_mesh = plsc.VectorSubcoreMesh(core_axis_name="c", subcore_axis_name="s")


def _wid():
    return lax.axis_index("s") * NC + lax.axis_index("c")


def _deg_body(edges_hbm, degp_hbm, packed_hbm, sbuf, dbuf, acc):
    wid = _wid()
    z16 = jnp.zeros((L,), jnp.float32)

    def zbody(i, c):
        acc[pl.ds(i * L, L)] = z16
        return c

    lax.fori_loop(0, N // L, zbody, 0)
    pltpu.sync_copy(edges_hbm.at[pl.ds(wid * EPW, EPW)], sbuf)
    pltpu.sync_copy(edges_hbm.at[pl.ds(E + wid * EPW, EPW)], dbuf)
    ones = jnp.ones((L,), jnp.float32)

    @plsc.parallel_loop(0, EPW, step=L)
    def _(i):
        s = sbuf[pl.ds(i, L)]
        d = dbuf[pl.ds(i, L)]
        plsc.addupdate_scatter(acc, [s], ones)
        plsc.addupdate_scatter(acc, [d], ones)
        sbuf[pl.ds(i, L)] = d * PACK + s

    pltpu.sync_copy(sbuf, packed_hbm.at[pl.ds(wid * EPW, EPW)])
    pltpu.sync_copy(acc, degp_hbm.at[pl.ds(wid * N, N)])


_deg_kernel = functools.partial(
    pl.kernel,
    out_type=[
        jax.ShapeDtypeStruct((NW * N,), jnp.float32),
        jax.ShapeDtypeStruct((E,), jnp.int32),
    ],
    mesh=_sc_mesh,
    compiler_params=pltpu.CompilerParams(needs_layout_passes=False),
    scratch_types=[
        pltpu.VMEM((EPW,), jnp.int32),
        pltpu.VMEM((EPW,), jnp.int32),
        pltpu.VMEM((N,), jnp.float32),
    ],
)(_deg_body)


def _mom_body(hT_hbm, t1T_hbm, packed_hbm, out_hbm, eb0, eb1, hc0, hc1, tc0,
              tc1, acc, sem0, sem1):
    wid = _wid()
    c0 = 2 * wid
    pltpu.sync_copy(hT_hbm.at[pl.ds(c0 * N, N)], hc0)
    pltpu.sync_copy(hT_hbm.at[pl.ds((c0 + 1) * N, N)], hc1)
    pltpu.sync_copy(t1T_hbm.at[pl.ds(c0 * N, N)], tc0)
    pltpu.sync_copy(t1T_hbm.at[pl.ds((c0 + 1) * N, N)], tc1)
    z16 = jnp.zeros((L,), jnp.float32)

    @plsc.parallel_loop(0, 6 * N, step=L)
    def _(i):
        acc[pl.ds(i, L)] = z16

    def make_body(ebuf):
        def body(i):
            p = ebuf[pl.ds(i, L)]
            s = jnp.bitwise_and(p, PACK - 1)
            d = jnp.right_shift(p, 14)
            a0 = plsc.load_gather(tc0, [s])
            a1 = plsc.load_gather(tc1, [s])
            g0 = plsc.load_gather(hc0, [s])
            g1 = plsc.load_gather(hc1, [s])
            plsc.addupdate_scatter(acc, [d], a0)
            plsc.addupdate_scatter(acc, [d + N], a1)
            a0 = a0 * g0
            a1 = a1 * g1
            plsc.addupdate_scatter(acc, [d + 2 * N], a0)
            plsc.addupdate_scatter(acc, [d + 3 * N], a1)
            a0 = a0 * g0
            a1 = a1 * g1
            plsc.addupdate_scatter(acc, [d + 4 * N], a0)
            plsc.addupdate_scatter(acc, [d + 5 * N], a1)

        return body

    bufs = (eb0, eb1)
    sems = (sem0, sem1)
    nblk = E // EB
    copies = [None, None]
    copies[0] = pltpu.async_copy(packed_hbm.at[pl.ds(0, EB)], bufs[0], sems[0])
    for blk in range(nblk):
        b = blk % 2
        copies[b].wait()
        if blk + 1 < nblk:
            copies[1 - b] = pltpu.async_copy(
                packed_hbm.at[pl.ds((blk + 1) * EB, EB)], bufs[1 - b],
                sems[1 - b])
        plsc.parallel_loop(0, EB, step=L, unroll=8)(make_body(bufs[b]))

    for pw in range(3):
        pltpu.sync_copy(acc.at[pl.ds((2 * pw) * N, N)],
                        out_hbm.at[pl.ds((pw * NH + c0) * N, N)])
        pltpu.sync_copy(acc.at[pl.ds((2 * pw + 1) * N, N)],
                        out_hbm.at[pl.ds((pw * NH + c0 + 1) * N, N)])


_mom_kernel = functools.partial(
    pl.kernel,
    out_type=jax.ShapeDtypeStruct((3 * NH * N,), jnp.float32),
    mesh=_sc_mesh,
    compiler_params=pltpu.CompilerParams(needs_layout_passes=False),
    scratch_types=[
        pltpu.VMEM((EB,), jnp.int32),
        pltpu.VMEM((EB,), jnp.int32),
        pltpu.VMEM((N,), jnp.float32),
        pltpu.VMEM((N,), jnp.float32),
        pltpu.VMEM((N,), jnp.float32),
        pltpu.VMEM((N,), jnp.float32),
        pltpu.VMEM((6 * N,), jnp.float32),
        pltpu.SemaphoreType.DMA,
        pltpu.SemaphoreType.DMA,
    ],
)(_mom_body)


def _pre_body(x_ref, w_ref, b_ref, degp_ref, hT_ref, t1T_ref, rs_ref):
    deg = jnp.sum(degp_ref[...], axis=0, keepdims=True)
    rs = lax.rsqrt(jnp.clip(deg, 1.0, None))
    rs_ref[...] = rs
    z = lax.dot_general(w_ref[...], x_ref[...], (((0,), (1,)), ((), ())),
                        preferred_element_type=jnp.float32)
    h = jnp.maximum(z + b_ref[...], 0.0)
    hT_ref[...] = h
    t1T_ref[...] = rs * h


def _pre_kernel(x, fc0_W, fc0_b, degp):
    return pl.pallas_call(
        _pre_body,
        out_shape=[
            jax.ShapeDtypeStruct((NH, N), jnp.float32),
            jax.ShapeDtypeStruct((NH, N), jnp.float32),
            jax.ShapeDtypeStruct((1, N), jnp.float32),
        ],
    )(x, fc0_W, fc0_b.reshape(NH, 1), degp)


def _layer_body(theta, last, momT_ref, rs_ref, hT_ref, h0T_ref, wa_ref,
                wt_ref, fwt_ref, fb_ref, out_ref, t1T_ref):
    rs = rs_ref[...]
    momT = momT_ref[...] * rs
    mu = momT[0:NH, :]
    s2 = momT[NH:2 * NH, :]
    s3 = momT[2 * NH:3 * NH, :]
    sigma = jnp.sqrt(jnp.clip(s2 - mu * mu, 1e-6, None))
    m3 = jnp.sign(s3) * jnp.power(jnp.abs(s3) + 1e-6, 1.0 / 3.0)
    hT = hT_ref[...]
    wa = wa_ref[...]
    waQ = wa[:, :NH]
    waK = wa[:, NH:]
    qT = jnp.dot(waQ, hT, preferred_element_type=jnp.float32)
    k0 = jnp.dot(waK[0:1, :], mu, preferred_element_type=jnp.float32)
    k1 = jnp.dot(waK[1:2, :], sigma, preferred_element_type=jnp.float32)
    k2 = jnp.dot(waK[2:3, :], m3, preferred_element_type=jnp.float32)
    a = jnp.tanh(qT + jnp.concatenate([k0, k1, k2], axis=0))
    a = a - jnp.max(a, axis=0, keepdims=True)
    ea = jnp.exp(a)
    attn = ea / jnp.sum(ea, axis=0, keepdims=True)
    aggT = attn[0:1, :] * mu + attn[1:2, :] * sigma + attn[2:3, :] * m3
    supportT = (1.0 - ALPHA) * aggT + ALPHA * h0T_ref[...]
    zT = jnp.dot(wt_ref[...], supportT, preferred_element_type=jnp.float32)
    h = jnp.maximum(theta * zT + (1.0 - theta) * supportT, 0.0)
    if last:
        lg = jnp.dot(fwt_ref[...], h, preferred_element_type=jnp.float32)
        lg = lg + fb_ref[...]
        lg = lg - jnp.max(lg, axis=0, keepdims=True)
        out_ref[...] = lg - jnp.log(jnp.sum(jnp.exp(lg), axis=0,
                                            keepdims=True))
        t1T_ref[...] = jnp.zeros_like(t1T_ref)
    else:
        out_ref[...] = h
        t1T_ref[...] = rs * h


def _layer_kernel(l, momT, rs, hT, h0T, wa, wt, fwt, fb):
    theta = math.log(LAMDA / (l + 1) + 1.0)
    last = l == NLAYERS - 1
    odim = NCLASS if last else NH
    return pl.pallas_call(
        functools.partial(_layer_body, theta, last),
        out_shape=[
            jax.ShapeDtypeStruct((odim, N), jnp.float32),
            jax.ShapeDtypeStruct((1, 1) if last else (NH, N), jnp.float32),
        ],
    )(momT, rs, hT, h0T, wa, wt, fwt, fb)


def kernel(x, edge_index, fc0_W, fc0_b, conv_weight, conv_watt, fc1_W, fc1_b):
    degp_flat, packed = _deg_kernel(edge_index.reshape(2 * E))
    degp = degp_flat.reshape(NW, N)
    hT, t1T, rs = _pre_kernel(x, fc0_W, fc0_b, degp)
    h0T = hT
    fwt = fc1_W.T
    fb = fc1_b.reshape(NCLASS, 1)
    for l in range(NLAYERS):
        momT = _mom_kernel(hT.reshape(NH * N), t1T.reshape(NH * N), packed)
        hT, t1T = _layer_kernel(l, momT.reshape(3 * NH, N), rs, hT, h0T,
                                conv_watt[l], conv_weight[l].T, fwt, fb)
    return hT.T


# speedup vs baseline: 22.0029x; 1.0580x over previous
import functools
import math

import jax
import jax.numpy as jnp
from jax import lax
from jax.experimental import pallas as pl
from jax.experimental.pallas import tpu as pltpu
from jax.experimental.pallas import tpu_sc as plsc

N = 10000
E = 320000
NFEAT = 128
NH = 64
NCLASS = 40
NLAYERS = 2
MOMENT = 3
LAMDA = 0.5
ALPHA = 0.1

NC = 2
NS = 16
L = 16
NW = NC * NS

BN = 1000
EPW = E // NW
EB = 10000
PACK = 16384

_sc_mesh = plsc.VectorSubcoreMesh(core_axis_name="c", subcore_axis_name="s")


def _wid():
    return lax.axis_index("s") * NC + lax.axis_index("c")


def _deg_body(edges_hbm, degp_hbm, packed_hbm, sbuf, dbuf, acc):
    wid = _wid()
    z16 = jnp.zeros((L,), jnp.float32)

    def zbody(i, c):
        acc[pl.ds(i * L, L)] = z16
        return c

    lax.fori_loop(0, N // L, zbody, 0)
    pltpu.sync_copy(edges_hbm.at[pl.ds(wid * EPW, EPW)], sbuf)
    pltpu.sync_copy(edges_hbm.at[pl.ds(E + wid * EPW, EPW)], dbuf)
    ones = jnp.ones((L,), jnp.float32)

    @plsc.parallel_loop(0, EPW, step=L)
    def _(i):
        s = sbuf[pl.ds(i, L)]
        d = dbuf[pl.ds(i, L)]
        plsc.addupdate_scatter(acc, [s], ones)
        plsc.addupdate_scatter(acc, [d], ones)
        sbuf[pl.ds(i, L)] = d * PACK + s

    pltpu.sync_copy(sbuf, packed_hbm.at[pl.ds(wid * EPW, EPW)])
    pltpu.sync_copy(acc, degp_hbm.at[pl.ds(wid * N, N)])


_deg_kernel = functools.partial(
    pl.kernel,
    out_type=[
        jax.ShapeDtypeStruct((NW * N,), jnp.float32),
        jax.ShapeDtypeStruct((E,), jnp.int32),
    ],
    mesh=_sc_mesh,
    compiler_params=pltpu.CompilerParams(needs_layout_passes=False),
    scratch_types=[
        pltpu.VMEM((EPW,), jnp.int32),
        pltpu.VMEM((EPW,), jnp.int32),
        pltpu.VMEM((N,), jnp.float32),
    ],
)(_deg_body)


def _mom_body(hb_hbm, t1T_hbm, packed_hbm, out_hbm, eb0, eb1, hcb, tc0,
              tc1, acc, sem0, sem1):
    wid = _wid()
    pltpu.sync_copy(hb_hbm.at[pl.ds(wid * N, N)], hcb)
    pltpu.sync_copy(t1T_hbm.at[pl.ds(wid * N, N)], tc0)
    pltpu.sync_copy(t1T_hbm.at[pl.ds((wid + 32) * N, N)], tc1)
    z16 = jnp.zeros((L,), jnp.float32)

    @plsc.parallel_loop(0, 6 * N, step=L)
    def _(i):
        acc[pl.ds(i, L)] = z16

    def make_body(ebuf):
        def body(i):
            p = ebuf[pl.ds(i, L)]
            s = jnp.bitwise_and(p, PACK - 1)
            d = jnp.right_shift(p, 14)
            a0 = plsc.load_gather(tc0, [s])
            a1 = plsc.load_gather(tc1, [s])
            gw = plsc.load_gather(hcb, [s])
            g0, g1 = plsc.unpack(plsc.bitcast(gw, jnp.bfloat16),
                                 format=plsc.PackFormat.INTERLEAVED)
            plsc.addupdate_scatter(acc, [d], a0)
            plsc.addupdate_scatter(acc, [d + N], a1)
            a0 = a0 * g0
            a1 = a1 * g1
            plsc.addupdate_scatter(acc, [d + 2 * N], a0)
            plsc.addupdate_scatter(acc, [d + 3 * N], a1)
            a0 = a0 * g0
            a1 = a1 * g1
            plsc.addupdate_scatter(acc, [d + 4 * N], a0)
            plsc.addupdate_scatter(acc, [d + 5 * N], a1)

        return body

    bufs = (eb0, eb1)
    sems = (sem0, sem1)
    nblk = E // EB
    copies = [None, None]
    copies[0] = pltpu.async_copy(packed_hbm.at[pl.ds(0, EB)], bufs[0], sems[0])
    for blk in range(nblk):
        b = blk % 2
        copies[b].wait()
        if blk + 1 < nblk:
            copies[1 - b] = pltpu.async_copy(
                packed_hbm.at[pl.ds((blk + 1) * EB, EB)], bufs[1 - b],
                sems[1 - b])
        plsc.parallel_loop(0, EB, step=L, unroll=8)(make_body(bufs[b]))

    for pw in range(3):
        pltpu.sync_copy(acc.at[pl.ds((2 * pw) * N, N)],
                        out_hbm.at[pl.ds((pw * NH + wid) * N, N)])
        pltpu.sync_copy(acc.at[pl.ds((2 * pw + 1) * N, N)],
                        out_hbm.at[pl.ds((pw * NH + wid + 32) * N, N)])


_mom_kernel = functools.partial(
    pl.kernel,
    out_type=jax.ShapeDtypeStruct((3 * NH * N,), jnp.float32),
    mesh=_sc_mesh,
    compiler_params=pltpu.CompilerParams(needs_layout_passes=False),
    scratch_types=[
        pltpu.VMEM((EB,), jnp.int32),
        pltpu.VMEM((EB,), jnp.int32),
        pltpu.VMEM((N,), jnp.int32),
        pltpu.VMEM((N,), jnp.float32),
        pltpu.VMEM((N,), jnp.float32),
        pltpu.VMEM((6 * N,), jnp.float32),
        pltpu.SemaphoreType.DMA,
        pltpu.SemaphoreType.DMA,
    ],
)(_mom_body)


def _pack_h(h):
    lo = lax.bitcast_convert_type(h[:NH // 2, :].astype(jnp.bfloat16),
                                  jnp.uint16).astype(jnp.uint32)
    hi = lax.bitcast_convert_type(h[NH // 2:, :].astype(jnp.bfloat16),
                                  jnp.uint16).astype(jnp.uint32)
    return lax.bitcast_convert_type(jnp.left_shift(hi, 16) | lo, jnp.int32)


def _pre_body(x_ref, w_ref, b_ref, degp_ref, hT_ref, hb_ref, t1T_ref, rs_ref):
    deg = jnp.sum(degp_ref[...], axis=0, keepdims=True)
    rs = lax.rsqrt(jnp.clip(deg, 1.0, None))
    rs_ref[...] = rs
    z = lax.dot_general(w_ref[...], x_ref[...], (((0,), (1,)), ((), ())),
                        preferred_element_type=jnp.float32)
    h = jnp.maximum(z + b_ref[...], 0.0)
    hT_ref[...] = h
    hb_ref[...] = _pack_h(h)
    t1T_ref[...] = rs * h


def _pre_kernel(x, fc0_W, fc0_b, degp):
    return pl.pallas_call(
        _pre_body,
        out_shape=[
            jax.ShapeDtypeStruct((NH, N), jnp.float32),
            jax.ShapeDtypeStruct((NH // 2, N), jnp.int32),
            jax.ShapeDtypeStruct((NH, N), jnp.float32),
            jax.ShapeDtypeStruct((1, N), jnp.float32),
        ],
    )(x, fc0_W, fc0_b.reshape(NH, 1), degp)


def _layer_body(theta, last, momT_ref, rs_ref, hT_ref, h0T_ref, wa_ref,
                wt_ref, fwt_ref, fb_ref, out_ref, hb_ref, t1T_ref):
    rs = rs_ref[...]
    momT = momT_ref[...] * rs
    mu = momT[0:NH, :]
    s2 = momT[NH:2 * NH, :]
    s3 = momT[2 * NH:3 * NH, :]
    sigma = jnp.sqrt(jnp.clip(s2 - mu * mu, 1e-6, None))
    m3 = jnp.sign(s3) * jnp.power(jnp.abs(s3) + 1e-6, 1.0 / 3.0)
    hT = hT_ref[...]
    wa = wa_ref[...]
    waQ = wa[:, :NH]
    waK = wa[:, NH:]
    qT = jnp.dot(waQ, hT, preferred_element_type=jnp.float32)
    k0 = jnp.dot(waK[0:1, :], mu, preferred_element_type=jnp.float32)
    k1 = jnp.dot(waK[1:2, :], sigma, preferred_element_type=jnp.float32)
    k2 = jnp.dot(waK[2:3, :], m3, preferred_element_type=jnp.float32)
    a = jnp.tanh(qT + jnp.concatenate([k0, k1, k2], axis=0))
    a = a - jnp.max(a, axis=0, keepdims=True)
    ea = jnp.exp(a)
    attn = ea / jnp.sum(ea, axis=0, keepdims=True)
    aggT = attn[0:1, :] * mu + attn[1:2, :] * sigma + attn[2:3, :] * m3
    supportT = (1.0 - ALPHA) * aggT + ALPHA * h0T_ref[...]
    zT = jnp.dot(wt_ref[...], supportT, preferred_element_type=jnp.float32)
    h = jnp.maximum(theta * zT + (1.0 - theta) * supportT, 0.0)
    if last:
        lg = jnp.dot(fwt_ref[...], h, preferred_element_type=jnp.float32)
        lg = lg + fb_ref[...]
        lg = lg - jnp.max(lg, axis=0, keepdims=True)
        out_ref[...] = lg - jnp.log(jnp.sum(jnp.exp(lg), axis=0,
                                            keepdims=True))
        hb_ref[...] = jnp.zeros_like(hb_ref)
        t1T_ref[...] = jnp.zeros_like(t1T_ref)
    else:
        out_ref[...] = h
        hb_ref[...] = _pack_h(h)
        t1T_ref[...] = rs * h


def _layer_kernel(l, momT, rs, hT, h0T, wa, wt, fwt, fb):
    theta = math.log(LAMDA / (l + 1) + 1.0)
    last = l == NLAYERS - 1
    odim = NCLASS if last else NH
    return pl.pallas_call(
        functools.partial(_layer_body, theta, last),
        out_shape=[
            jax.ShapeDtypeStruct((odim, N), jnp.float32),
            jax.ShapeDtypeStruct((1, 1) if last else (NH // 2, N), jnp.int32),
            jax.ShapeDtypeStruct((1, 1) if last else (NH, N), jnp.float32),
        ],
    )(momT, rs, hT, h0T, wa, wt, fwt, fb)


def kernel(x, edge_index, fc0_W, fc0_b, conv_weight, conv_watt, fc1_W, fc1_b):
    degp_flat, packed = _deg_kernel(edge_index.reshape(2 * E))
    degp = degp_flat.reshape(NW, N)
    hT, hb, t1T, rs = _pre_kernel(x, fc0_W, fc0_b, degp)
    h0T = hT
    fwt = fc1_W.T
    fb = fc1_b.reshape(NCLASS, 1)
    for l in range(NLAYERS):
        momT = _mom_kernel(hb.reshape(NH // 2 * N), t1T.reshape(NH * N),
                           packed)
        hT, hb, t1T = _layer_kernel(l, momT.reshape(3 * NH, N), rs, hT, h0T,
                                    conv_watt[l], conv_weight[l].T, fwt, fb)
    return hT.T


# speedup vs baseline: 23.2075x; 1.0547x over previous
import functools
import math

import jax
import jax.numpy as jnp
from jax import lax
from jax.experimental import pallas as pl
from jax.experimental.pallas import tpu as pltpu
from jax.experimental.pallas import tpu_sc as plsc

N = 10000
E = 320000
NFEAT = 128
NH = 64
NCLASS = 40
NLAYERS = 2
MOMENT = 3
LAMDA = 0.5
ALPHA = 0.1

NC = 2
NS = 16
L = 16
NW = NC * NS

BN = 1000
EPW = E // NW
EB = 10000
PACK = 16384

_sc_mesh = plsc.VectorSubcoreMesh(core_axis_name="c", subcore_axis_name="s")


def _wid():
    return lax.axis_index("s") * NC + lax.axis_index("c")


def _deg_body(edges_hbm, degp_hbm, packed_hbm, sbuf, dbuf, acc):
    wid = _wid()
    z16 = jnp.zeros((L,), jnp.float32)

    def zbody(i, c):
        acc[pl.ds(i * L, L)] = z16
        return c

    lax.fori_loop(0, N // L, zbody, 0)
    pltpu.sync_copy(edges_hbm.at[pl.ds(wid * EPW, EPW)], sbuf)
    pltpu.sync_copy(edges_hbm.at[pl.ds(E + wid * EPW, EPW)], dbuf)
    ones = jnp.ones((L,), jnp.float32)

    @plsc.parallel_loop(0, EPW, step=L)
    def _(i):
        s = sbuf[pl.ds(i, L)]
        d = dbuf[pl.ds(i, L)]
        plsc.addupdate_scatter(acc, [s], ones)
        plsc.addupdate_scatter(acc, [d], ones)
        sbuf[pl.ds(i, L)] = d * PACK + s

    pltpu.sync_copy(sbuf, packed_hbm.at[pl.ds(wid * EPW, EPW)])
    pltpu.sync_copy(acc, degp_hbm.at[pl.ds(wid * N, N)])


_deg_kernel = functools.partial(
    pl.kernel,
    out_type=[
        jax.ShapeDtypeStruct((NW * N,), jnp.float32),
        jax.ShapeDtypeStruct((E,), jnp.int32),
    ],
    mesh=_sc_mesh,
    compiler_params=pltpu.CompilerParams(needs_layout_passes=False),
    scratch_types=[
        pltpu.VMEM((EPW,), jnp.int32),
        pltpu.VMEM((EPW,), jnp.int32),
        pltpu.VMEM((N,), jnp.float32),
    ],
)(_deg_body)


def _mom_body(hb_hbm, tb_hbm, packed_hbm, out_hbm, eb0, eb1, hcb, tcb,
              acc, sem0, sem1):
    wid = _wid()
    pltpu.sync_copy(hb_hbm.at[pl.ds(wid * N, N)], hcb)
    pltpu.sync_copy(tb_hbm.at[pl.ds(wid * N, N)], tcb)
    z16 = jnp.zeros((L,), jnp.float32)

    @plsc.parallel_loop(0, 6 * N, step=L)
    def _(i):
        acc[pl.ds(i, L)] = z16

    def make_body(ebuf):
        def body(i):
            p = ebuf[pl.ds(i, L)]
            s = jnp.bitwise_and(p, PACK - 1)
            d = jnp.right_shift(p, 14)
            tw = plsc.load_gather(tcb, [s])
            a0, a1 = plsc.unpack(plsc.bitcast(tw, jnp.bfloat16),
                                 format=plsc.PackFormat.INTERLEAVED)
            gw = plsc.load_gather(hcb, [s])
            g0, g1 = plsc.unpack(plsc.bitcast(gw, jnp.bfloat16),
                                 format=plsc.PackFormat.INTERLEAVED)
            plsc.addupdate_scatter(acc, [d], a0)
            plsc.addupdate_scatter(acc, [d + N], a1)
            a0 = a0 * g0
            a1 = a1 * g1
            plsc.addupdate_scatter(acc, [d + 2 * N], a0)
            plsc.addupdate_scatter(acc, [d + 3 * N], a1)
            a0 = a0 * g0
            a1 = a1 * g1
            plsc.addupdate_scatter(acc, [d + 4 * N], a0)
            plsc.addupdate_scatter(acc, [d + 5 * N], a1)

        return body

    bufs = (eb0, eb1)
    sems = (sem0, sem1)
    nblk = E // EB
    copies = [None, None]
    copies[0] = pltpu.async_copy(packed_hbm.at[pl.ds(0, EB)], bufs[0], sems[0])
    for blk in range(nblk):
        b = blk % 2
        copies[b].wait()
        if blk + 1 < nblk:
            copies[1 - b] = pltpu.async_copy(
                packed_hbm.at[pl.ds((blk + 1) * EB, EB)], bufs[1 - b],
                sems[1 - b])
        plsc.parallel_loop(0, EB, step=L, unroll=8)(make_body(bufs[b]))

    for pw in range(3):
        pltpu.sync_copy(acc.at[pl.ds((2 * pw) * N, N)],
                        out_hbm.at[pl.ds((pw * NH + wid) * N, N)])
        pltpu.sync_copy(acc.at[pl.ds((2 * pw + 1) * N, N)],
                        out_hbm.at[pl.ds((pw * NH + wid + 32) * N, N)])


_mom_kernel = functools.partial(
    pl.kernel,
    out_type=jax.ShapeDtypeStruct((3 * NH * N,), jnp.float32),
    mesh=_sc_mesh,
    compiler_params=pltpu.CompilerParams(needs_layout_passes=False),
    scratch_types=[
        pltpu.VMEM((EB,), jnp.int32),
        pltpu.VMEM((EB,), jnp.int32),
        pltpu.VMEM((N,), jnp.int32),
        pltpu.VMEM((N,), jnp.int32),
        pltpu.VMEM((6 * N,), jnp.float32),
        pltpu.SemaphoreType.DMA,
        pltpu.SemaphoreType.DMA,
    ],
)(_mom_body)


def _pack_h(h):
    lo = lax.bitcast_convert_type(h[:NH // 2, :].astype(jnp.bfloat16),
                                  jnp.uint16).astype(jnp.uint32)
    hi = lax.bitcast_convert_type(h[NH // 2:, :].astype(jnp.bfloat16),
                                  jnp.uint16).astype(jnp.uint32)
    return lax.bitcast_convert_type(jnp.left_shift(hi, 16) | lo, jnp.int32)


def _pre_body(x_ref, w_ref, b_ref, degp_ref, hT_ref, hb_ref, tb_ref, rs_ref):
    deg = jnp.sum(degp_ref[...], axis=0, keepdims=True)
    rs = lax.rsqrt(jnp.clip(deg, 1.0, None))
    rs_ref[...] = rs
    z = lax.dot_general(w_ref[...], x_ref[...], (((0,), (1,)), ((), ())),
                        preferred_element_type=jnp.float32)
    h = jnp.maximum(z + b_ref[...], 0.0)
    hT_ref[...] = h
    hb_ref[...] = _pack_h(h)
    tb_ref[...] = _pack_h(rs * h)


def _pre_kernel(x, fc0_W, fc0_b, degp):
    return pl.pallas_call(
        _pre_body,
        out_shape=[
            jax.ShapeDtypeStruct((NH, N), jnp.float32),
            jax.ShapeDtypeStruct((NH // 2, N), jnp.int32),
            jax.ShapeDtypeStruct((NH // 2, N), jnp.int32),
            jax.ShapeDtypeStruct((1, N), jnp.float32),
        ],
    )(x, fc0_W, fc0_b.reshape(NH, 1), degp)


def _layer_body(theta, last, momT_ref, rs_ref, hT_ref, h0T_ref, wa_ref,
                wt_ref, fwt_ref, fb_ref, out_ref, hb_ref, tb_ref):
    rs = rs_ref[...]
    momT = momT_ref[...] * rs
    mu = momT[0:NH, :]
    s2 = momT[NH:2 * NH, :]
    s3 = momT[2 * NH:3 * NH, :]
    sigma = jnp.sqrt(jnp.clip(s2 - mu * mu, 1e-6, None))
    m3 = jnp.sign(s3) * jnp.power(jnp.abs(s3) + 1e-6, 1.0 / 3.0)
    hT = hT_ref[...]
    wa = wa_ref[...]
    waQ = wa[:, :NH]
    waK = wa[:, NH:]
    qT = jnp.dot(waQ, hT, preferred_element_type=jnp.float32)
    k0 = jnp.dot(waK[0:1, :], mu, preferred_element_type=jnp.float32)
    k1 = jnp.dot(waK[1:2, :], sigma, preferred_element_type=jnp.float32)
    k2 = jnp.dot(waK[2:3, :], m3, preferred_element_type=jnp.float32)
    a = jnp.tanh(qT + jnp.concatenate([k0, k1, k2], axis=0))
    a = a - jnp.max(a, axis=0, keepdims=True)
    ea = jnp.exp(a)
    attn = ea / jnp.sum(ea, axis=0, keepdims=True)
    aggT = attn[0:1, :] * mu + attn[1:2, :] * sigma + attn[2:3, :] * m3
    supportT = (1.0 - ALPHA) * aggT + ALPHA * h0T_ref[...]
    zT = jnp.dot(wt_ref[...], supportT, preferred_element_type=jnp.float32)
    h = jnp.maximum(theta * zT + (1.0 - theta) * supportT, 0.0)
    if last:
        lg = jnp.dot(fwt_ref[...], h, preferred_element_type=jnp.float32)
        lg = lg + fb_ref[...]
        lg = lg - jnp.max(lg, axis=0, keepdims=True)
        out_ref[...] = lg - jnp.log(jnp.sum(jnp.exp(lg), axis=0,
                                            keepdims=True))
        hb_ref[...] = jnp.zeros_like(hb_ref)
        tb_ref[...] = jnp.zeros_like(tb_ref)
    else:
        out_ref[...] = h
        hb_ref[...] = _pack_h(h)
        tb_ref[...] = _pack_h(rs * h)


def _layer_kernel(l, momT, rs, hT, h0T, wa, wt, fwt, fb):
    theta = math.log(LAMDA / (l + 1) + 1.0)
    last = l == NLAYERS - 1
    odim = NCLASS if last else NH
    return pl.pallas_call(
        functools.partial(_layer_body, theta, last),
        out_shape=[
            jax.ShapeDtypeStruct((odim, N), jnp.float32),
            jax.ShapeDtypeStruct((1, 1) if last else (NH // 2, N), jnp.int32),
            jax.ShapeDtypeStruct((1, 1) if last else (NH // 2, N), jnp.int32),
        ],
    )(momT, rs, hT, h0T, wa, wt, fwt, fb)


def kernel(x, edge_index, fc0_W, fc0_b, conv_weight, conv_watt, fc1_W, fc1_b):
    degp_flat, packed = _deg_kernel(edge_index.reshape(2 * E))
    degp = degp_flat.reshape(NW, N)
    hT, hb, tb, rs = _pre_kernel(x, fc0_W, fc0_b, degp)
    h0T = hT
    fwt = fc1_W.T
    fb = fc1_b.reshape(NCLASS, 1)
    for l in range(NLAYERS):
        momT = _mom_kernel(hb.reshape(NH // 2 * N), tb.reshape(NH // 2 * N),
                           packed)
        hT, hb, tb = _layer_kernel(l, momT.reshape(3 * NH, N), rs, hT, h0T,
                                   conv_watt[l], conv_weight[l].T, fwt, fb)
    return hT.T


# speedup vs baseline: 23.7305x; 1.0225x over previous
import functools
import math

import jax
import jax.numpy as jnp
from jax import lax
from jax.experimental import pallas as pl
from jax.experimental.pallas import tpu as pltpu
from jax.experimental.pallas import tpu_sc as plsc

N = 10000
E = 320000
NFEAT = 128
NH = 64
NCLASS = 40
NLAYERS = 2
MOMENT = 3
LAMDA = 0.5
ALPHA = 0.1

NC = 2
NS = 16
L = 16
NW = NC * NS

BN = 1000
EPW = E // NW
EB = 20000
PACK = 16384

_sc_mesh = plsc.VectorSubcoreMesh(core_axis_name="c", subcore_axis_name="s")


def _wid():
    return lax.axis_index("s") * NC + lax.axis_index("c")


def _deg_body(edges_hbm, degp_hbm, packed_hbm, sbuf, dbuf, acc):
    wid = _wid()
    z16 = jnp.zeros((L,), jnp.float32)

    def zbody(i, c):
        acc[pl.ds(i * L, L)] = z16
        return c

    lax.fori_loop(0, N // L, zbody, 0)
    pltpu.sync_copy(edges_hbm.at[pl.ds(wid * EPW, EPW)], sbuf)
    pltpu.sync_copy(edges_hbm.at[pl.ds(E + wid * EPW, EPW)], dbuf)
    ones = jnp.ones((L,), jnp.float32)

    @plsc.parallel_loop(0, EPW, step=L)
    def _(i):
        s = sbuf[pl.ds(i, L)]
        d = dbuf[pl.ds(i, L)]
        plsc.addupdate_scatter(acc, [s], ones)
        plsc.addupdate_scatter(acc, [d], ones)
        sbuf[pl.ds(i, L)] = d * PACK + s

    pltpu.sync_copy(sbuf, packed_hbm.at[pl.ds(wid * EPW, EPW)])
    pltpu.sync_copy(acc, degp_hbm.at[pl.ds(wid * N, N)])


_deg_kernel = functools.partial(
    pl.kernel,
    out_type=[
        jax.ShapeDtypeStruct((NW * N,), jnp.float32),
        jax.ShapeDtypeStruct((E,), jnp.int32),
    ],
    mesh=_sc_mesh,
    compiler_params=pltpu.CompilerParams(needs_layout_passes=False),
    scratch_types=[
        pltpu.VMEM((EPW,), jnp.int32),
        pltpu.VMEM((EPW,), jnp.int32),
        pltpu.VMEM((N,), jnp.float32),
    ],
)(_deg_body)


def _mom_body(hb_hbm, tb_hbm, packed_hbm, out_hbm, eb0, eb1, hcb, tcb,
              acc, sem0, sem1):
    wid = _wid()
    pltpu.sync_copy(hb_hbm.at[pl.ds(wid * N, N)], hcb)
    pltpu.sync_copy(tb_hbm.at[pl.ds(wid * N, N)], tcb)
    z16 = jnp.zeros((L,), jnp.float32)

    @plsc.parallel_loop(0, 6 * N, step=L)
    def _(i):
        acc[pl.ds(i, L)] = z16

    def make_body(ebuf):
        def body(i):
            p = ebuf[pl.ds(i, L)]
            s = jnp.bitwise_and(p, PACK - 1)
            d = jnp.right_shift(p, 14)
            tw = plsc.load_gather(tcb, [s])
            a0, a1 = plsc.unpack(plsc.bitcast(tw, jnp.bfloat16),
                                 format=plsc.PackFormat.INTERLEAVED)
            gw = plsc.load_gather(hcb, [s])
            g0, g1 = plsc.unpack(plsc.bitcast(gw, jnp.bfloat16),
                                 format=plsc.PackFormat.INTERLEAVED)
            plsc.addupdate_scatter(acc, [d], a0)
            plsc.addupdate_scatter(acc, [d + N], a1)
            a0 = a0 * g0
            a1 = a1 * g1
            plsc.addupdate_scatter(acc, [d + 2 * N], a0)
            plsc.addupdate_scatter(acc, [d + 3 * N], a1)
            a0 = a0 * g0
            a1 = a1 * g1
            plsc.addupdate_scatter(acc, [d + 4 * N], a0)
            plsc.addupdate_scatter(acc, [d + 5 * N], a1)

        return body

    bufs = (eb0, eb1)
    sems = (sem0, sem1)
    nblk = E // EB
    copies = [None, None]
    copies[0] = pltpu.async_copy(packed_hbm.at[pl.ds(0, EB)], bufs[0], sems[0])
    for blk in range(nblk):
        b = blk % 2
        copies[b].wait()
        if blk + 1 < nblk:
            copies[1 - b] = pltpu.async_copy(
                packed_hbm.at[pl.ds((blk + 1) * EB, EB)], bufs[1 - b],
                sems[1 - b])
        plsc.parallel_loop(0, EB, step=L, unroll=16)(make_body(bufs[b]))

    for pw in range(3):
        pltpu.sync_copy(acc.at[pl.ds((2 * pw) * N, N)],
                        out_hbm.at[pl.ds((pw * NH + wid) * N, N)])
        pltpu.sync_copy(acc.at[pl.ds((2 * pw + 1) * N, N)],
                        out_hbm.at[pl.ds((pw * NH + wid + 32) * N, N)])


_mom_kernel = functools.partial(
    pl.kernel,
    out_type=jax.ShapeDtypeStruct((3 * NH * N,), jnp.float32),
    mesh=_sc_mesh,
    compiler_params=pltpu.CompilerParams(needs_layout_passes=False),
    scratch_types=[
        pltpu.VMEM((EB,), jnp.int32),
        pltpu.VMEM((EB,), jnp.int32),
        pltpu.VMEM((N,), jnp.int32),
        pltpu.VMEM((N,), jnp.int32),
        pltpu.VMEM((6 * N,), jnp.float32),
        pltpu.SemaphoreType.DMA,
        pltpu.SemaphoreType.DMA,
    ],
)(_mom_body)


def _pack_h(h):
    lo = lax.bitcast_convert_type(h[:NH // 2, :].astype(jnp.bfloat16),
                                  jnp.uint16).astype(jnp.uint32)
    hi = lax.bitcast_convert_type(h[NH // 2:, :].astype(jnp.bfloat16),
                                  jnp.uint16).astype(jnp.uint32)
    return lax.bitcast_convert_type(jnp.left_shift(hi, 16) | lo, jnp.int32)


def _pre_body(x_ref, w_ref, b_ref, degp_ref, hT_ref, hb_ref, tb_ref, rs_ref):
    deg = jnp.sum(degp_ref[...], axis=0, keepdims=True)
    rs = lax.rsqrt(jnp.clip(deg, 1.0, None))
    rs_ref[...] = rs
    z = lax.dot_general(w_ref[...], x_ref[...], (((0,), (1,)), ((), ())),
                        preferred_element_type=jnp.float32)
    h = jnp.maximum(z + b_ref[...], 0.0)
    hT_ref[...] = h
    hb_ref[...] = _pack_h(h)
    tb_ref[...] = _pack_h(rs * h)


def _pre_kernel(x, fc0_W, fc0_b, degp):
    return pl.pallas_call(
        _pre_body,
        out_shape=[
            jax.ShapeDtypeStruct((NH, N), jnp.float32),
            jax.ShapeDtypeStruct((NH // 2, N), jnp.int32),
            jax.ShapeDtypeStruct((NH // 2, N), jnp.int32),
            jax.ShapeDtypeStruct((1, N), jnp.float32),
        ],
    )(x, fc0_W, fc0_b.reshape(NH, 1), degp)


def _layer_body(theta, last, momT_ref, rs_ref, hT_ref, h0T_ref, wa_ref,
                wt_ref, fwt_ref, fb_ref, out_ref, hb_ref, tb_ref):
    rs = rs_ref[...]
    momT = momT_ref[...] * rs
    mu = momT[0:NH, :]
    s2 = momT[NH:2 * NH, :]
    s3 = momT[2 * NH:3 * NH, :]
    sigma = jnp.sqrt(jnp.clip(s2 - mu * mu, 1e-6, None))
    m3 = jnp.sign(s3) * jnp.power(jnp.abs(s3) + 1e-6, 1.0 / 3.0)
    hT = hT_ref[...]
    wa = wa_ref[...]
    waQ = wa[:, :NH]
    waK = wa[:, NH:]
    qT = jnp.dot(waQ, hT, preferred_element_type=jnp.float32)
    k0 = jnp.dot(waK[0:1, :], mu, preferred_element_type=jnp.float32)
    k1 = jnp.dot(waK[1:2, :], sigma, preferred_element_type=jnp.float32)
    k2 = jnp.dot(waK[2:3, :], m3, preferred_element_type=jnp.float32)
    a = jnp.tanh(qT + jnp.concatenate([k0, k1, k2], axis=0))
    a = a - jnp.max(a, axis=0, keepdims=True)
    ea = jnp.exp(a)
    attn = ea / jnp.sum(ea, axis=0, keepdims=True)
    aggT = attn[0:1, :] * mu + attn[1:2, :] * sigma + attn[2:3, :] * m3
    supportT = (1.0 - ALPHA) * aggT + ALPHA * h0T_ref[...]
    zT = jnp.dot(wt_ref[...], supportT, preferred_element_type=jnp.float32)
    h = jnp.maximum(theta * zT + (1.0 - theta) * supportT, 0.0)
    if last:
        lg = jnp.dot(fwt_ref[...], h, preferred_element_type=jnp.float32)
        lg = lg + fb_ref[...]
        lg = lg - jnp.max(lg, axis=0, keepdims=True)
        out_ref[...] = lg - jnp.log(jnp.sum(jnp.exp(lg), axis=0,
                                            keepdims=True))
        hb_ref[...] = jnp.zeros_like(hb_ref)
        tb_ref[...] = jnp.zeros_like(tb_ref)
    else:
        out_ref[...] = h
        hb_ref[...] = _pack_h(h)
        tb_ref[...] = _pack_h(rs * h)


def _layer_kernel(l, momT, rs, hT, h0T, wa, wt, fwt, fb):
    theta = math.log(LAMDA / (l + 1) + 1.0)
    last = l == NLAYERS - 1
    odim = NCLASS if last else NH
    return pl.pallas_call(
        functools.partial(_layer_body, theta, last),
        out_shape=[
            jax.ShapeDtypeStruct((odim, N), jnp.float32),
            jax.ShapeDtypeStruct((1, 1) if last else (NH // 2, N), jnp.int32),
            jax.ShapeDtypeStruct((1, 1) if last else (NH // 2, N), jnp.int32),
        ],
    )(momT, rs, hT, h0T, wa, wt, fwt, fb)


def kernel(x, edge_index, fc0_W, fc0_b, conv_weight, conv_watt, fc1_W, fc1_b):
    degp_flat, packed = _deg_kernel(edge_index.reshape(2 * E))
    degp = degp_flat.reshape(NW, N)
    hT, hb, tb, rs = _pre_kernel(x, fc0_W, fc0_b, degp)
    h0T = hT
    fwt = fc1_W.T
    fb = fc1_b.reshape(NCLASS, 1)
    for l in range(NLAYERS):
        momT = _mom_kernel(hb.reshape(NH // 2 * N), tb.reshape(NH // 2 * N),
                           packed)
        hT, hb, tb = _layer_kernel(l, momT.reshape(3 * NH, N), rs, hT, h0T,
                                   conv_watt[l], conv_weight[l].T, fwt, fb)
    return hT.T
